# baseline (speedup 1.0000x reference)
import functools

import jax
import jax.numpy as jnp
from jax import lax
from jax.experimental import pallas as pl
from jax.experimental.pallas import tpu as pltpu
from jax.experimental.pallas import tpu_sc as plsc

D_MODEL = 512
MAX_LEN = 12
BATCH = 4096
NUM_CORES = 2
NUM_SUBCORES = 16
NUM_WORKERS = NUM_CORES * NUM_SUBCORES
LENS_PER_WORKER = BATCH // NUM_WORKERS
ROWS_PER_WORKER = LENS_PER_WORKER * MAX_LEN
CHUNK_ROWS = 96
NUM_CHUNKS = ROWS_PER_WORKER // CHUNK_ROWS
LANES = 16
LEN_GROUPS = LENS_PER_WORKER // LANES


def _pe_body(lens_hbm, table_hbm, out_hbm, lens_v, idx_v, buf0, buf1, sem0, sem1):
    wid = lax.axis_index("s") * NUM_CORES + lax.axis_index("c")
    base_len = wid * LENS_PER_WORKER
    pltpu.sync_copy(lens_hbm.at[pl.ds(base_len, LENS_PER_WORKER)], lens_v)

    lane = lax.iota(jnp.int32, LANES)
    ge = [lane >= (MAX_LEN - 4 * v) for v in range(3)]
    j_vec = [lane + (4 * v) - jnp.where(ge[v], MAX_LEN, 0) for v in range(3)]
    j_vec1 = [j + 1 for j in j_vec]
    bstep = [jnp.where(ge[v], v + 1, v) for v in range(3)]
    for g in range(LEN_GROUPS):
        grp = lens_v[pl.ds(g * LANES, LANES)]
        for w in range(MAX_LEN):
            v = w % 3
            b_local = bstep[v] + 4 * (w // 3)
            len_vec = lax.gather(
                grp,
                b_local[:, None],
                lax.GatherDimensionNumbers(
                    offset_dims=(), collapsed_slice_dims=(0,), start_index_map=(0,)
                ),
                (1,),
                mode=lax.GatherScatterMode.PROMISE_IN_BOUNDS,
            )
            idx_vec = jnp.where(j_vec[v] < len_vec, j_vec1[v], 0)
            idx_v[pl.ds(g * 192 + w * LANES, LANES)] = idx_vec

    bufs = (buf0, buf1)
    sems = (sem0, sem1)
    row_base = wid * ROWS_PER_WORKER
    handles = [None, None]
    for c in range(NUM_CHUNKS):
        b = c % 2
        handles[b] = pltpu.async_copy(
            table_hbm.at[idx_v.at[pl.ds(c * CHUNK_ROWS, CHUNK_ROWS)]], bufs[b], sems[b]
        )
        if c > 0:
            handles[1 - b].wait()
            pltpu.sync_copy(
                bufs[1 - b],
                out_hbm.at[pl.ds(row_base + (c - 1) * CHUNK_ROWS, CHUNK_ROWS)],
            )
    last = (NUM_CHUNKS - 1) % 2
    handles[last].wait()
    pltpu.sync_copy(
        bufs[last],
        out_hbm.at[pl.ds(row_base + (NUM_CHUNKS - 1) * CHUNK_ROWS, CHUNK_ROWS)],
    )


_pe_call = functools.partial(
    pl.kernel,
    mesh=plsc.VectorSubcoreMesh(core_axis_name="c", subcore_axis_name="s"),
    out_type=jax.ShapeDtypeStruct((BATCH * MAX_LEN, D_MODEL), jnp.float32),
    scratch_types=[
        pltpu.VMEM((LENS_PER_WORKER,), jnp.int32),
        pltpu.VMEM((ROWS_PER_WORKER,), jnp.int32),
        pltpu.VMEM((CHUNK_ROWS, D_MODEL), jnp.float32),
        pltpu.VMEM((CHUNK_ROWS, D_MODEL), jnp.float32),
        pltpu.SemaphoreType.DMA,
        pltpu.SemaphoreType.DMA,
    ],
)(_pe_body)


def kernel(input_lens, table):
    out = _pe_call(input_lens, table)
    return out.reshape(BATCH, MAX_LEN, D_MODEL)


# speedup vs baseline: 2.7665x; 2.7665x over previous
import functools

import jax
import jax.numpy as jnp
from jax import lax
from jax.experimental import pallas as pl
from jax.experimental.pallas import tpu as pltpu
from jax.experimental.pallas import tpu_sc as plsc

D_MODEL = 512
MAX_LEN = 12
BATCH = 4096
NUM_CORES = 2
NUM_SUBCORES = 16
NUM_WORKERS = NUM_CORES * NUM_SUBCORES
LENS_PER_WORKER = BATCH // NUM_WORKERS
ROWS_PER_WORKER = LENS_PER_WORKER * MAX_LEN
CHUNK_ELEMS = 4
CHUNK_ROWS = CHUNK_ELEMS * MAX_LEN
CHUNK_WORDS = CHUNK_ROWS * D_MODEL
NUM_CHUNKS = ROWS_PER_WORKER // CHUNK_ROWS
LANES = 16
LEN_GROUPS = LENS_PER_WORKER // LANES
VREGS_PER_ROW = D_MODEL // LANES


def _pe_body(lens_hbm, pe_hbm, out_hbm, lens_v, mask_v, pe_v, stag2, sem0):
    wid = lax.axis_index("s") * NUM_CORES + lax.axis_index("c")
    base_len = wid * LENS_PER_WORKER
    pltpu.sync_copy(lens_hbm.at[pl.ds(base_len, LENS_PER_WORKER)], lens_v)
    pltpu.sync_copy(pe_hbm, pe_v)

    lane = lax.iota(jnp.int32, LANES)
    ge = [lane >= (MAX_LEN - 4 * v) for v in range(3)]
    j_vec = [lane + (4 * v) - jnp.where(ge[v], MAX_LEN, 0) for v in range(3)]
    bstep = [jnp.where(ge[v], v + 1, v) for v in range(3)]
    ones = jnp.full((LANES,), 1.0, jnp.float32)
    zeros = jnp.zeros((LANES,), jnp.float32)
    for g in range(LEN_GROUPS):
        grp = lens_v[pl.ds(g * LANES, LANES)]
        for w in range(MAX_LEN):
            v = w % 3
            b_local = bstep[v] + 4 * (w // 3)
            len_vec = lax.gather(
                grp,
                b_local[:, None],
                lax.GatherDimensionNumbers(
                    offset_dims=(), collapsed_slice_dims=(0,), start_index_map=(0,)
                ),
                (1,),
                mode=lax.GatherScatterMode.PROMISE_IN_BOUNDS,
            )
            mask_vec = jnp.where(j_vec[v] < len_vec, ones, zeros)
            mask_v[pl.ds(g * 192 + w * LANES, LANES)] = mask_vec

    word_base = wid * ROWS_PER_WORKER * D_MODEL

    gd = lax.GatherDimensionNumbers(
        offset_dims=(), collapsed_slice_dims=(0,), start_index_map=(0,)
    )
    splats = [lane * 0 + r for r in range(LANES)]

    def _drain_one():
        pltpu.make_async_copy(
            out_hbm.at[pl.ds(0, CHUNK_WORDS)], stag2.at[pl.ds(0, CHUNK_WORDS)], sem0
        ).wait()

    def chunk_body(c, carry):
        half = (c & 1) * CHUNK_WORDS

        @pl.when(c >= 2)
        def _():
            _drain_one()

        def group_body(t, carry2):
            mgrp = mask_v[pl.ds(c * CHUNK_ROWS + t * LANES, LANES)]
            for r in range(LANES):
                m = lax.gather(
                    mgrp,
                    splats[r][:, None],
                    gd,
                    (1,),
                    mode=lax.GatherScatterMode.PROMISE_IN_BOUNDS,
                )
                row_words = (t * LANES + r) * D_MODEL
                for k in range(VREGS_PER_ROW):
                    stag2[pl.ds(half + row_words + k * LANES, LANES)] = (
                        pe_v[pl.ds(row_words + k * LANES, LANES)] * m
                    )
            return carry2

        lax.fori_loop(0, CHUNK_ROWS // LANES, group_body, 0)
        pltpu.async_copy(
            stag2.at[pl.ds(half, CHUNK_WORDS)],
            out_hbm.at[pl.ds(word_base + c * CHUNK_WORDS, CHUNK_WORDS)],
            sem0,
        )
        return carry

    lax.fori_loop(0, NUM_CHUNKS, chunk_body, 0)
    _drain_one()
    _drain_one()


_pe_call = functools.partial(
    pl.kernel,
    mesh=plsc.VectorSubcoreMesh(core_axis_name="c", subcore_axis_name="s"),
    out_type=jax.ShapeDtypeStruct((BATCH * MAX_LEN * D_MODEL,), jnp.float32),
    scratch_types=[
        pltpu.VMEM((LENS_PER_WORKER,), jnp.int32),
        pltpu.VMEM((ROWS_PER_WORKER,), jnp.float32),
        pltpu.VMEM((CHUNK_WORDS,), jnp.float32),
        pltpu.VMEM((2 * CHUNK_WORDS,), jnp.float32),
        pltpu.SemaphoreType.DMA,
    ],
)(_pe_body)


def kernel(input_lens, table):
    pe_flat = jnp.tile(lax.slice(table, (1, 0), (MAX_LEN + 1, D_MODEL)), (CHUNK_ELEMS, 1)).reshape(-1)
    out = _pe_call(input_lens, pe_flat)
    return out.reshape(BATCH, MAX_LEN, D_MODEL)


# speedup vs baseline: 5.0225x; 1.8155x over previous
import functools

import jax
import jax.numpy as jnp
from jax import lax
from jax.experimental import pallas as pl
from jax.experimental.pallas import tpu as pltpu
from jax.experimental.pallas import tpu_sc as plsc

D_MODEL = 512
MAX_LEN = 12
BATCH = 4096
NUM_CORES = 2
NUM_SUBCORES = 16
NUM_WORKERS = NUM_CORES * NUM_SUBCORES
LENS_PER_WORKER = BATCH // NUM_WORKERS
ROWS_PER_WORKER = LENS_PER_WORKER * MAX_LEN
CHUNK_ELEMS = 2
CHUNK_ROWS = CHUNK_ELEMS * MAX_LEN
CHUNK_WORDS = CHUNK_ROWS * D_MODEL
NUM_CHUNKS = ROWS_PER_WORKER // CHUNK_ROWS
LANES = 16
LEN_GROUPS = LENS_PER_WORKER // LANES
VREGS_PER_ROW = D_MODEL // LANES


def _pe_body(lens_hbm, pe_hbm, out_hbm, lens_v, mask_v, pe_v, stag2, sem0):
    wid = lax.axis_index("s") * NUM_CORES + lax.axis_index("c")
    base_len = wid * LENS_PER_WORKER
    pltpu.sync_copy(lens_hbm.at[pl.ds(base_len, LENS_PER_WORKER)], lens_v)
    pltpu.sync_copy(pe_hbm, pe_v)

    lane = lax.iota(jnp.int32, LANES)
    ge = [lane >= (MAX_LEN - 4 * v) for v in range(3)]
    j_vec = [lane + (4 * v) - jnp.where(ge[v], MAX_LEN, 0) for v in range(3)]
    bstep = [jnp.where(ge[v], v + 1, v) for v in range(3)]
    ones = jnp.full((LANES,), 1.0, jnp.float32)
    zeros = jnp.zeros((LANES,), jnp.float32)
    for g in range(LEN_GROUPS):
        grp = lens_v[pl.ds(g * LANES, LANES)]
        for w in range(MAX_LEN):
            v = w % 3
            b_local = bstep[v] + 4 * (w // 3)
            len_vec = lax.gather(
                grp,
                b_local[:, None],
                lax.GatherDimensionNumbers(
                    offset_dims=(), collapsed_slice_dims=(0,), start_index_map=(0,)
                ),
                (1,),
                mode=lax.GatherScatterMode.PROMISE_IN_BOUNDS,
            )
            mask_vec = jnp.where(j_vec[v] < len_vec, ones, zeros)
            mask_v[pl.ds(g * 192 + w * LANES, LANES)] = mask_vec

    word_base = wid * ROWS_PER_WORKER * D_MODEL

    gd = lax.GatherDimensionNumbers(
        offset_dims=(), collapsed_slice_dims=(0,), start_index_map=(0,)
    )
    splats = [lane * 0 + r for r in range(LANES)]

    def _drain_one():
        pltpu.make_async_copy(
            out_hbm.at[pl.ds(0, CHUNK_WORDS)], stag2.at[pl.ds(0, CHUNK_WORDS)], sem0
        ).wait()

    def chunk_body(c, carry):
        @pl.when(c >= 2)
        def _():
            _drain_one()

        def do_half(half_words):
            mg0 = mask_v[pl.ds(c * CHUNK_ROWS, LANES)]
            mg1 = mask_v[pl.ds(c * CHUNK_ROWS + LANES, LANES)]
            for q in range(CHUNK_ROWS):
                grp, idx = (mg0, q) if q < LANES else (mg1, q - LANES)
                m = lax.gather(
                    grp,
                    splats[idx][:, None],
                    gd,
                    (1,),
                    mode=lax.GatherScatterMode.PROMISE_IN_BOUNDS,
                )
                row_words = q * D_MODEL
                for k in range(VREGS_PER_ROW):
                    stag2[pl.ds(half_words + row_words + k * LANES, LANES)] = (
                        pe_v[pl.ds(row_words + k * LANES, LANES)] * m
                    )
            pltpu.async_copy(
                stag2.at[pl.ds(half_words, CHUNK_WORDS)],
                out_hbm.at[pl.ds(word_base + c * CHUNK_WORDS, CHUNK_WORDS)],
                sem0,
            )

        @pl.when((c & 1) == 0)
        def _():
            do_half(0)

        @pl.when((c & 1) == 1)
        def _():
            do_half(CHUNK_WORDS)

        return carry

    lax.fori_loop(0, NUM_CHUNKS, chunk_body, 0)
    _drain_one()
    _drain_one()


_pe_call = functools.partial(
    pl.kernel,
    mesh=plsc.VectorSubcoreMesh(core_axis_name="c", subcore_axis_name="s"),
    out_type=jax.ShapeDtypeStruct((BATCH * MAX_LEN * D_MODEL,), jnp.float32),
    scratch_types=[
        pltpu.VMEM((LENS_PER_WORKER,), jnp.int32),
        pltpu.VMEM((ROWS_PER_WORKER + 2 * LANES,), jnp.float32),
        pltpu.VMEM((CHUNK_WORDS,), jnp.float32),
        pltpu.VMEM((2 * CHUNK_WORDS,), jnp.float32),
        pltpu.SemaphoreType.DMA,
    ],
)(_pe_body)


def kernel(input_lens, table):
    pe_flat = jnp.tile(lax.slice(table, (1, 0), (MAX_LEN + 1, D_MODEL)), (CHUNK_ELEMS, 1)).reshape(-1)
    out = _pe_call(input_lens, pe_flat)
    return out.reshape(BATCH, MAX_LEN, D_MODEL)


# speedup vs baseline: 7.9367x; 1.5802x over previous
import functools

import jax
import jax.numpy as jnp
from jax import lax
from jax.experimental import pallas as pl
from jax.experimental.pallas import tpu as pltpu
from jax.experimental.pallas import tpu_sc as plsc

D_MODEL = 512
MAX_LEN = 12
BATCH = 4096
NUM_CORES = 2
NUM_SUBCORES = 16
NUM_WORKERS = NUM_CORES * NUM_SUBCORES
LENS_PER_WORKER = BATCH // NUM_WORKERS
ROWS_PER_WORKER = LENS_PER_WORKER * MAX_LEN
CHUNK_ELEMS = 2
CHUNK_ROWS = CHUNK_ELEMS * MAX_LEN
CHUNK_WORDS = CHUNK_ROWS * D_MODEL
NUM_CHUNKS = ROWS_PER_WORKER // CHUNK_ROWS
LANES = 16
LEN_GROUPS = LENS_PER_WORKER // LANES
VREGS_PER_ROW = D_MODEL // LANES


def _pe_body(lens_hbm, pe_hbm, out_hbm, lens_v, mask_v, pe_v, stag2, sem0):
    wid = lax.axis_index("s") * NUM_CORES + lax.axis_index("c")
    base_len = wid * LENS_PER_WORKER
    pltpu.sync_copy(lens_hbm.at[pl.ds(base_len, LENS_PER_WORKER)], lens_v)
    pltpu.sync_copy(pe_hbm, pe_v)

    lane = lax.iota(jnp.int32, LANES)
    ge = [lane >= (MAX_LEN - 4 * v) for v in range(3)]
    j_vec = [lane + (4 * v) - jnp.where(ge[v], MAX_LEN, 0) for v in range(3)]
    bstep = [jnp.where(ge[v], v + 1, v) for v in range(3)]
    ones = jnp.full((LANES,), 1.0, jnp.float32)
    zeros = jnp.zeros((LANES,), jnp.float32)
    for g in range(LEN_GROUPS):
        grp = lens_v[pl.ds(g * LANES, LANES)]
        for w in range(MAX_LEN):
            v = w % 3
            b_local = bstep[v] + 4 * (w // 3)
            len_vec = lax.gather(
                grp,
                b_local[:, None],
                lax.GatherDimensionNumbers(
                    offset_dims=(), collapsed_slice_dims=(0,), start_index_map=(0,)
                ),
                (1,),
                mode=lax.GatherScatterMode.PROMISE_IN_BOUNDS,
            )
            mask_vec = jnp.where(j_vec[v] < len_vec, ones, zeros)
            mask_v[pl.ds(g * 192 + w * LANES, LANES)] = mask_vec

    elem_base = wid * LENS_PER_WORKER

    gd = lax.GatherDimensionNumbers(
        offset_dims=(), collapsed_slice_dims=(0,), start_index_map=(0,)
    )
    splats = [lane * 0 + r for r in range(LANES)]

    def _drain_one():
        pltpu.make_async_copy(
            out_hbm.at[pl.ds(0, CHUNK_ELEMS)], stag2.at[pl.ds(0, CHUNK_ELEMS)], sem0
        ).wait()

    def chunk_body(c, carry):
        @pl.when(c >= 2)
        def _():
            _drain_one()

        def do_half(half_elems):
            mg0 = mask_v[pl.ds(c * CHUNK_ROWS, LANES)]
            mg1 = mask_v[pl.ds(c * CHUNK_ROWS + LANES, LANES)]
            for q in range(CHUNK_ROWS):
                grp, idx = (mg0, q) if q < LANES else (mg1, q - LANES)
                m = lax.gather(
                    grp,
                    splats[idx][:, None],
                    gd,
                    (1,),
                    mode=lax.GatherScatterMode.PROMISE_IN_BOUNDS,
                )
                e, j = divmod(q, MAX_LEN)
                row_words = q * D_MODEL
                for k in range(VREGS_PER_ROW):
                    stag2[half_elems + e, j, pl.ds(k * LANES, LANES)] = (
                        pe_v[pl.ds(row_words + k * LANES, LANES)] * m
                    )
            pltpu.async_copy(
                stag2.at[pl.ds(half_elems, CHUNK_ELEMS)],
                out_hbm.at[pl.ds(elem_base + c * CHUNK_ELEMS, CHUNK_ELEMS)],
                sem0,
            )

        @pl.when((c & 1) == 0)
        def _():
            do_half(0)

        @pl.when((c & 1) == 1)
        def _():
            do_half(CHUNK_ELEMS)

        return carry

    lax.fori_loop(0, NUM_CHUNKS, chunk_body, 0)
    _drain_one()
    _drain_one()


_pe_call = functools.partial(
    pl.kernel,
    mesh=plsc.VectorSubcoreMesh(core_axis_name="c", subcore_axis_name="s"),
    out_type=jax.ShapeDtypeStruct((BATCH, MAX_LEN, D_MODEL), jnp.float32),
    scratch_types=[
        pltpu.VMEM((LENS_PER_WORKER,), jnp.int32),
        pltpu.VMEM((ROWS_PER_WORKER + 2 * LANES,), jnp.float32),
        pltpu.VMEM((CHUNK_WORDS,), jnp.float32),
        pltpu.VMEM((2 * CHUNK_ELEMS, MAX_LEN, D_MODEL), jnp.float32),
        pltpu.SemaphoreType.DMA,
    ],
    compiler_params=pltpu.CompilerParams(use_tc_tiling_on_sc=True),
)(_pe_body)


def kernel(input_lens, table):
    pe_flat = jnp.tile(lax.slice(table, (1, 0), (MAX_LEN + 1, D_MODEL)), (CHUNK_ELEMS, 1)).reshape(-1)
    return _pe_call(input_lens, pe_flat)


# speedup vs baseline: 7.9710x; 1.0043x over previous
import functools

import jax
import jax.numpy as jnp
from jax import lax
from jax.experimental import pallas as pl
from jax.experimental.pallas import tpu as pltpu
from jax.experimental.pallas import tpu_sc as plsc

D_MODEL = 512
MAX_LEN = 12
BATCH = 4096
NUM_CORES = 2
NUM_SUBCORES = 16
NUM_WORKERS = NUM_CORES * NUM_SUBCORES
LENS_PER_WORKER = BATCH // NUM_WORKERS
ROWS_PER_WORKER = LENS_PER_WORKER * MAX_LEN
CHUNK_ELEMS = 2
CHUNK_ROWS = CHUNK_ELEMS * MAX_LEN
CHUNK_WORDS = CHUNK_ROWS * D_MODEL
NUM_CHUNKS = ROWS_PER_WORKER // CHUNK_ROWS
LANES = 16
LEN_GROUPS = LENS_PER_WORKER // LANES
VREGS_PER_ROW = D_MODEL // LANES


def _pe_body(lens_hbm, table_hbm, out_hbm, lens_v, mask_v, pe_v, stag2, sem0):
    wid = lax.axis_index("s") * NUM_CORES + lax.axis_index("c")
    base_len = wid * LENS_PER_WORKER
    pltpu.sync_copy(lens_hbm.at[pl.ds(base_len, LENS_PER_WORKER)], lens_v)
    pltpu.sync_copy(table_hbm.at[pl.ds(0, LANES)], pe_v)

    lane = lax.iota(jnp.int32, LANES)
    ge = [lane >= (MAX_LEN - 4 * v) for v in range(3)]
    j_vec = [lane + (4 * v) - jnp.where(ge[v], MAX_LEN, 0) for v in range(3)]
    bstep = [jnp.where(ge[v], v + 1, v) for v in range(3)]
    ones = jnp.full((LANES,), 1.0, jnp.float32)
    zeros = jnp.zeros((LANES,), jnp.float32)
    for g in range(LEN_GROUPS):
        grp = lens_v[pl.ds(g * LANES, LANES)]
        for w in range(MAX_LEN):
            v = w % 3
            b_local = bstep[v] + 4 * (w // 3)
            len_vec = lax.gather(
                grp,
                b_local[:, None],
                lax.GatherDimensionNumbers(
                    offset_dims=(), collapsed_slice_dims=(0,), start_index_map=(0,)
                ),
                (1,),
                mode=lax.GatherScatterMode.PROMISE_IN_BOUNDS,
            )
            mask_vec = jnp.where(j_vec[v] < len_vec, ones, zeros)
            mask_v[pl.ds(g * 192 + w * LANES, LANES)] = mask_vec

    elem_base = wid * LENS_PER_WORKER

    gd = lax.GatherDimensionNumbers(
        offset_dims=(), collapsed_slice_dims=(0,), start_index_map=(0,)
    )
    splats = [lane * 0 + r for r in range(LANES)]

    def _drain_one():
        pltpu.make_async_copy(
            out_hbm.at[pl.ds(0, CHUNK_ELEMS)], stag2.at[pl.ds(0, CHUNK_ELEMS)], sem0
        ).wait()

    def chunk_body(c, carry):
        @pl.when(c >= 2)
        def _():
            _drain_one()

        def do_half(half_elems):
            mg0 = mask_v[pl.ds(c * CHUNK_ROWS, LANES)]
            mg1 = mask_v[pl.ds(c * CHUNK_ROWS + LANES, LANES)]
            for q in range(CHUNK_ROWS):
                grp, idx = (mg0, q) if q < LANES else (mg1, q - LANES)
                m = lax.gather(
                    grp,
                    splats[idx][:, None],
                    gd,
                    (1,),
                    mode=lax.GatherScatterMode.PROMISE_IN_BOUNDS,
                )
                e, j = divmod(q, MAX_LEN)
                for k in range(VREGS_PER_ROW):
                    stag2[half_elems + e, j, pl.ds(k * LANES, LANES)] = (
                        pe_v[j + 1, pl.ds(k * LANES, LANES)] * m
                    )
            pltpu.async_copy(
                stag2.at[pl.ds(half_elems, CHUNK_ELEMS)],
                out_hbm.at[pl.ds(elem_base + c * CHUNK_ELEMS, CHUNK_ELEMS)],
                sem0,
            )

        @pl.when((c & 1) == 0)
        def _():
            do_half(0)

        @pl.when((c & 1) == 1)
        def _():
            do_half(CHUNK_ELEMS)

        return carry

    lax.fori_loop(0, NUM_CHUNKS, chunk_body, 0)
    _drain_one()
    _drain_one()


_pe_call = functools.partial(
    pl.kernel,
    mesh=plsc.VectorSubcoreMesh(core_axis_name="c", subcore_axis_name="s"),
    out_type=jax.ShapeDtypeStruct((BATCH, MAX_LEN, D_MODEL), jnp.float32),
    scratch_types=[
        pltpu.VMEM((LENS_PER_WORKER,), jnp.int32),
        pltpu.VMEM((ROWS_PER_WORKER + 2 * LANES,), jnp.float32),
        pltpu.VMEM((LANES, D_MODEL), jnp.float32),
        pltpu.VMEM((2 * CHUNK_ELEMS, MAX_LEN, D_MODEL), jnp.float32),
        pltpu.SemaphoreType.DMA,
    ],
    compiler_params=pltpu.CompilerParams(use_tc_tiling_on_sc=True),
)(_pe_body)


def kernel(input_lens, table):
    return _pe_call(input_lens, table)
